# speedup vs baseline: 1.5358x; 1.5358x over previous
import functools

import jax
import jax.numpy as jnp
from jax import lax
from jax.experimental import pallas as pl
from jax.experimental.pallas import tpu as pltpu
from jax.experimental.pallas import tpu_sc as plsc

N = 10000
E = 160000
NODE_IN = 128
EDGE_IN = 16
LATENT = 128
NODE_OUT = 3
EDGE_OUT = 1
STEPS = 4

NP = 10240
EP = 163840
NCORES = 2
NSUB = 16
NW = NCORES * NSUB
EPW = EP // NW
CHUNK = 128
NCH = EPW // CHUNK
ROWS_PER_TILE = NP // NSUB

_TILE = 512


def _mm(a, b):
    return jax.lax.dot_general(a, b, (((1,), (0,)), ((), ())),
                               preferred_element_type=jnp.float32)


def _mlp_tail(z1, w2, b2, w3, b3, g, be, ln):
    h1 = jnp.maximum(z1, 0.0)
    z2 = _mm(h1, w2[...]) + b2[...]
    h2 = jnp.maximum(z2, 0.0)
    z3 = _mm(h2, w3[...]) + b3[...]
    if ln:
        mu = jnp.mean(z3, axis=-1, keepdims=True)
        var = jnp.mean(jnp.square(z3 - mu), axis=-1, keepdims=True)
        z3 = (z3 - mu) * jax.lax.rsqrt(var + 1e-5) * g[...] + be[...]
    return z3


def _enc_body(x, w1, b1, w2, b2, w3, b3, g, be, o_ref):
    z1 = _mm(x[...], w1[...]) + b1[...]
    o_ref[...] = _mlp_tail(z1, w2, b2, w3, b3, g, be, True)


def _edge_body(hs, hd, e, w1a, w1b, w1c, b1, w2, b2, w3, b3, g, be,
               enew_ref, enext_ref):
    z1 = (_mm(hs[...], w1a[...]) + _mm(hd[...], w1b[...])
          + _mm(e[...], w1c[...]) + b1[...])
    en = _mlp_tail(z1, w2, b2, w3, b3, g, be, True)
    enew_ref[...] = en
    enext_ref[...] = e[...] + en


def _node_body(h, a0, a1, w1a, w1b, b1, w2, b2, w3, b3, g, be, o_ref):
    z1 = (_mm(h[...], w1a[...]) + _mm(a0[...] + a1[...], w1b[...]) + b1[...])
    o_ref[...] = h[...] + _mlp_tail(z1, w2, b2, w3, b3, g, be, True)


def _dec_body(x, w1, b1, w2, b2, w3, b3, o_ref):
    z1 = _mm(x[...], w1[...]) + b1[...]
    o_ref[...] = _mlp_tail(z1, w2, b2, w3, b3, None, None, False)


def _row_spec(width):
    return pl.BlockSpec((_TILE, width), lambda r: (r, 0))


def _fix_spec(shape):
    return pl.BlockSpec(shape, lambda r: (0,) * len(shape))


def _mlp_call(body, rows, in_arrays, n_row_args, n_out):
    grid = rows // _TILE
    in_specs = []
    for i, a in enumerate(in_arrays):
        if i < n_row_args:
            in_specs.append(_row_spec(a.shape[1]))
        else:
            in_specs.append(_fix_spec(a.shape))
    out_shape = [jax.ShapeDtypeStruct((rows, LATENT), jnp.float32)
                 for _ in range(n_out)]
    out_specs = [_row_spec(LATENT) for _ in range(n_out)]
    if n_out == 1:
        out_shape, out_specs = out_shape[0], out_specs[0]
    return pl.pallas_call(
        body,
        grid=(grid,),
        in_specs=in_specs,
        out_specs=out_specs,
        out_shape=out_shape,
        compiler_params=pltpu.CompilerParams(
            dimension_semantics=("arbitrary",)),
    )(*in_arrays)


_MESH = dict(core_axis_name="c", subcore_axis_name="s")


def _sc_gather_fn():
    mesh = plsc.VectorSubcoreMesh(**_MESH)

    @functools.partial(
        pl.kernel,
        out_type=(jax.ShapeDtypeStruct((EP, LATENT), jnp.float32),
                  jax.ShapeDtypeStruct((EP, LATENT), jnp.float32)),
        mesh=mesh,
        scratch_types=[
            pltpu.VMEM((CHUNK,), jnp.int32),
            pltpu.VMEM((CHUNK,), jnp.int32),
            pltpu.VMEM((CHUNK, LATENT), jnp.float32),
            pltpu.VMEM((CHUNK, LATENT), jnp.float32),
            pltpu.SemaphoreType.DMA,
            pltpu.SemaphoreType.DMA,
        ],
    )
    def k(h_hbm, src_hbm, dst_hbm, hs_hbm, hd_hbm,
          idx_s, idx_d, rows_s, rows_d, sem_s, sem_d):
        cid = lax.axis_index("c")
        sid = lax.axis_index("s")
        wid = sid * NCORES + cid
        base0 = wid * EPW

        def chunk(c, carry):
            base = pl.multiple_of(base0 + c * CHUNK, CHUNK)
            pltpu.sync_copy(src_hbm.at[pl.ds(base, CHUNK)], idx_s)
            pltpu.sync_copy(dst_hbm.at[pl.ds(base, CHUNK)], idx_d)
            cs = pltpu.async_copy(h_hbm.at[idx_s], rows_s, sem_s)
            cd = pltpu.async_copy(h_hbm.at[idx_d], rows_d, sem_d)
            cs.wait()
            pltpu.sync_copy(rows_s, hs_hbm.at[pl.ds(base, CHUNK)])
            cd.wait()
            pltpu.sync_copy(rows_d, hd_hbm.at[pl.ds(base, CHUNK)])
            return carry

        lax.fori_loop(0, NCH, chunk, 0)

    return k


def _sc_scatter_fn():
    mesh = plsc.VectorSubcoreMesh(**_MESH)

    @functools.partial(
        pl.kernel,
        out_type=jax.ShapeDtypeStruct((NCORES, NP, LATENT), jnp.float32),
        mesh=mesh,
        scratch_types=[
            pltpu.VMEM((CHUNK,), jnp.int32),
            pltpu.VMEM((CHUNK, LATENT), jnp.float32),
            pltpu.VMEM_SHARED((NP, LATENT), jnp.float32),
        ],
    )
    def k(enew_hbm, dst_hbm, zeros_hbm, agg_hbm, idx_v, rows_v, agg_sh):
        cid = lax.axis_index("c")
        sid = lax.axis_index("s")
        wid = sid * NCORES + cid
        base0 = wid * EPW
        rbase = pl.multiple_of(sid * ROWS_PER_TILE, CHUNK)

        pltpu.sync_copy(zeros_hbm.at[pl.ds(rbase, ROWS_PER_TILE)],
                        agg_sh.at[pl.ds(rbase, ROWS_PER_TILE)])
        plsc.subcore_barrier()

        def chunk(c, carry):
            base = pl.multiple_of(base0 + c * CHUNK, CHUNK)
            pltpu.sync_copy(dst_hbm.at[pl.ds(base, CHUNK)], idx_v)
            pltpu.sync_copy(enew_hbm.at[pl.ds(base, CHUNK)], rows_v)
            pltpu.sync_copy(rows_v, agg_sh.at[idx_v], add=True)
            return carry

        lax.fori_loop(0, NCH, chunk, 0)
        plsc.subcore_barrier()
        pltpu.sync_copy(agg_sh.at[pl.ds(rbase, ROWS_PER_TILE)],
                        agg_hbm.at[cid, pl.ds(rbase, ROWS_PER_TILE)])

    return k


def _b(v):
    return v.reshape(1, -1)


def kernel(x, edge_index, edge_features, params):
    src = edge_index[0]
    dst = edge_index[1]
    x_p = jnp.pad(x, ((0, NP - N), (0, 0)))
    ef_p = jnp.pad(edge_features, ((0, EP - E), (0, 0)))
    src_p = jnp.pad(src, (0, EP - E))
    dst_p = jnp.pad(dst, (0, EP - E), constant_values=N)
    zeros_np = jnp.zeros((NP, LATENT), jnp.float32)

    pe = params['enc_node']
    h = _mlp_call(_enc_body, NP,
                  [x_p, pe['W'][0], _b(pe['b'][0]), pe['W'][1], _b(pe['b'][1]),
                   pe['W'][2], _b(pe['b'][2]), _b(pe['g']), _b(pe['be'])],
                  1, 1)
    pe = params['enc_edge']
    e = _mlp_call(_enc_body, EP,
                  [ef_p, pe['W'][0], _b(pe['b'][0]), pe['W'][1], _b(pe['b'][1]),
                   pe['W'][2], _b(pe['b'][2]), _b(pe['g']), _b(pe['be'])],
                  1, 1)

    gather = _sc_gather_fn()
    scatter = _sc_scatter_fn()

    for blk in params['proc']:
        hs, hd = gather(h, src_p, dst_p)
        pb = blk['edge']
        w1 = pb['W'][0]
        e_new, e = _mlp_call(
            _edge_body, EP,
            [hs, hd, e,
             w1[:LATENT], w1[LATENT:2 * LATENT], w1[2 * LATENT:],
             _b(pb['b'][0]), pb['W'][1], _b(pb['b'][1]), pb['W'][2],
             _b(pb['b'][2]), _b(pb['g']), _b(pb['be'])],
            3, 2)
        agg2 = scatter(e_new, dst_p, zeros_np)
        pb = blk['node']
        w1 = pb['W'][0]
        h = _mlp_call(
            _node_body, NP,
            [h, agg2[0], agg2[1],
             w1[:LATENT], w1[LATENT:],
             _b(pb['b'][0]), pb['W'][1], _b(pb['b'][1]), pb['W'][2],
             _b(pb['b'][2]), _b(pb['g']), _b(pb['be'])],
            3, 1)

    pd = params['dec_node']
    w3 = jnp.pad(pd['W'][2], ((0, 0), (0, LATENT - NODE_OUT)))
    b3 = jnp.pad(pd['b'][2], (0, LATENT - NODE_OUT))
    out_node = _mlp_call(
        _dec_body, NP,
        [h, pd['W'][0], _b(pd['b'][0]), pd['W'][1], _b(pd['b'][1]),
         w3, _b(b3)],
        1, 1)[:N, :NODE_OUT]
    pd = params['dec_edge']
    w3 = jnp.pad(pd['W'][2], ((0, 0), (0, LATENT - EDGE_OUT)))
    b3 = jnp.pad(pd['b'][2], (0, LATENT - EDGE_OUT))
    out_edge = _mlp_call(
        _dec_body, EP,
        [e, pd['W'][0], _b(pd['b'][0]), pd['W'][1], _b(pd['b'][1]),
         w3, _b(b3)],
        1, 1)[:E, :EDGE_OUT]
    return (out_node, out_edge)
